# baseline (speedup 1.0000x reference)
import functools

import jax
import jax.numpy as jnp
from jax import lax
from jax.experimental import pallas as pl
from jax.experimental.pallas import tpu as pltpu
from jax.experimental.pallas import tpu_sc as plsc

_CHUNK = 80
_LANES = 16


def _make_argmax(b, n, vocab, n_workers):
    bpw = b // n_workers
    full = vocab // _LANES
    tail = vocab - full * _LANES

    def _argmax_body(x_hbm, tok_hbm, slab_v, tok_v, sem):
        wid = lax.axis_index("s") * 2 + lax.axis_index("c")
        base = wid * bpw
        lane = lax.iota(jnp.int32, _LANES)

        pltpu.async_copy(x_hbm.at[base], slab_v.at[0], sem)

        def slab_step(s, _):
            par = lax.rem(s, 2)
            pltpu.make_async_copy(x_hbm.at[base + s], slab_v.at[par], sem).wait()

            @pl.when(s + 1 < bpw)
            def _():
                pltpu.async_copy(
                    x_hbm.at[base + s + 1], slab_v.at[1 - par], sem)

            def row_step(r, _):
                best_v = slab_v[par, r, pl.ds(0, _LANES)]
                best_i = lane
                for j in range(1, full):
                    v = slab_v[par, r, pl.ds(j * _LANES, _LANES)]
                    upd = v > best_v
                    best_v = jnp.where(upd, v, best_v)
                    best_i = jnp.where(upd, lane + j * _LANES, best_i)
                if tail:
                    v = slab_v[par, r, pl.ds(vocab - _LANES, _LANES)]
                    upd = v > best_v
                    best_v = jnp.where(upd, v, best_v)
                    best_i = jnp.where(upd, lane + (vocab - _LANES), best_i)
                m = jnp.max(best_v)
                tok = jnp.min(jnp.where(best_v == m, best_i, vocab))
                plsc.store_scatter(
                    tok_v,
                    [jnp.full((_LANES,), s * n + r, jnp.int32)],
                    jnp.full((_LANES,), tok, jnp.int32),
                    mask=lane == 0,
                )
                return 0

            lax.fori_loop(0, n, row_step, 0, unroll=False)
            return 0

        lax.fori_loop(0, bpw, slab_step, 0, unroll=False)
        pltpu.sync_copy(tok_v, tok_hbm.at[pl.ds(wid * bpw * n, bpw * n)])

    mesh = plsc.VectorSubcoreMesh(core_axis_name="c", subcore_axis_name="s")
    return pl.kernel(
        _argmax_body,
        mesh=mesh,
        compiler_params=pltpu.CompilerParams(needs_layout_passes=False),
        out_type=jax.ShapeDtypeStruct((b * n,), jnp.int32),
        scratch_types=[
            pltpu.VMEM((2, n, vocab), jnp.float32),
            pltpu.VMEM((bpw * n,), jnp.int32),
            pltpu.SemaphoreType.DMA,
        ],
    )


def _make_gather(rows, emb, n_workers, n_chunks):
    bpw = rows // n_workers

    def _gather_body(tok_hbm, table_hbm, out_hbm, idx_v, rows_v, sem):
        wid = lax.axis_index("s") * 2 + lax.axis_index("c")
        pltpu.sync_copy(tok_hbm.at[pl.ds(wid * bpw, bpw)], idx_v)
        copies = [
            pltpu.async_copy(
                table_hbm.at[idx_v.at[pl.ds(j * _CHUNK, _CHUNK)]],
                rows_v.at[pl.ds(j * _CHUNK, _CHUNK)],
                sem,
            )
            for j in range(n_chunks)
        ]
        for cp in copies:
            cp.wait()
        pltpu.sync_copy(rows_v, out_hbm.at[pl.ds(wid * bpw, bpw)])

    mesh = plsc.VectorSubcoreMesh(core_axis_name="c", subcore_axis_name="s")
    return pl.kernel(
        _gather_body,
        mesh=mesh,
        compiler_params=pltpu.CompilerParams(use_tc_tiling_on_sc=False),
        out_type=jax.ShapeDtypeStruct((rows, emb), jnp.float32),
        scratch_types=[
            pltpu.VMEM((bpw,), jnp.int32),
            pltpu.VMEM((bpw, emb), jnp.float32),
            pltpu.SemaphoreType.DMA,
        ],
    )


def kernel(x, dictionary):
    b, n, vocab = x.shape
    emb = dictionary.shape[1]
    rows = b * n
    n_workers = 32
    n_chunks = rows // (n_workers * _CHUNK)

    tokens = _make_argmax(b, n, vocab, n_workers)(x)
    out = _make_gather(rows, emb, n_workers, n_chunks)(tokens, dictionary)
    return out.reshape(b, n, emb)
